# baseline (speedup 1.0000x reference)
import functools

import jax
import jax.numpy as jnp
from jax import lax
from jax.experimental import pallas as pl
from jax.experimental.pallas import tpu as pltpu
from jax.experimental.pallas import tpu_sc as plsc

N = 10000
E = 320000
H = 128
G = 64

NC = 2
NS = 16
NW = NC * NS
EPW = E // NW
CHUNK = 80
NCHUNK = EPW // CHUNK
NPAD = 10240
RPT = NPAD // NS


def _sc_aggregate(x, src_flat, dst, zeros):
    mesh = plsc.VectorSubcoreMesh(core_axis_name="c", subcore_axis_name="s")

    @functools.partial(
        pl.kernel,
        out_type=jax.ShapeDtypeStruct((NC, NPAD, H), jnp.float32),
        mesh=mesh,
        scratch_types=[
            pltpu.VMEM((EPW,), jnp.int32),
            pltpu.VMEM((NCHUNK, CHUNK), jnp.int32),
            pltpu.VMEM((CHUNK, H), jnp.float32),
            pltpu.VMEM((CHUNK, H), jnp.float32),
            pltpu.VMEM_SHARED((NPAD, H), jnp.float32),
            pltpu.SemaphoreType.DMA,
            pltpu.SemaphoreType.DMA,
        ],
    )
    def agg(x_hbm, src_hbm, dst_hbm, z_hbm, out_hbm,
            src_v, dst_v, rows_a, rows_b, acc_sh, sem_a, sem_b):
        c = lax.axis_index("c")
        s = lax.axis_index("s")
        wid = c * NS + s
        pltpu.sync_copy(src_hbm.at[pl.ds(wid * EPW, EPW)], src_v)
        pltpu.sync_copy(dst_hbm.at[wid], dst_v)
        pltpu.sync_copy(z_hbm.at[pl.ds(s * RPT, RPT)],
                        acc_sh.at[pl.ds(s * RPT, RPT)])
        plsc.subcore_barrier()

        def gather(j, rows, sem):
            pltpu.async_copy(
                x_hbm.at[src_v.at[pl.ds(j * CHUNK, CHUNK)]], rows, sem)

        def scat(j, rows, sem):
            pltpu.make_async_copy(
                x_hbm.at[src_v.at[pl.ds(j * CHUNK, CHUNK)]],
                rows, sem).wait()
            pltpu.sync_copy(rows, acc_sh.at[dst_v.at[j]], add=True)

        gather(0, rows_a, sem_a)

        def pair(j, issue_next):
            gather(j + 1, rows_b, sem_b)
            scat(j, rows_a, sem_a)
            if issue_next:
                gather(j + 2, rows_a, sem_a)
            scat(j + 1, rows_b, sem_b)

        def body(p, carry):
            pair(2 * p, True)
            return carry

        lax.fori_loop(0, NCHUNK // 2, body, 0)
        scat(NCHUNK - 1, rows_a, sem_a)
        plsc.subcore_barrier()
        pltpu.sync_copy(acc_sh.at[pl.ds(s * RPT, RPT)],
                        out_hbm.at[c, pl.ds(s * RPT, RPT)])

    return agg(x, src_flat, dst, zeros)


_BLK = 1000


def _tc_root(x, W, b):
    def body(x_ref, w_ref, b_ref, o_ref):
        o_ref[...] = (jnp.dot(x_ref[...], w_ref[...],
                              preferred_element_type=jnp.float32)
                      + b_ref[...])

    return pl.pallas_call(
        body,
        grid=(N // _BLK,),
        in_specs=[
            pl.BlockSpec((_BLK, H), lambda i: (i, 0)),
            pl.BlockSpec((H, H), lambda i: (0, 0)),
            pl.BlockSpec((1, H), lambda i: (0, 0)),
        ],
        out_specs=pl.BlockSpec((_BLK, H), lambda i: (i, 0)),
        out_shape=jax.ShapeDtypeStruct((N, H), jnp.float32),
    )(x, W, b)


def _tc_mid(p, xroot1, W_rel1, W_rel2):
    def body(p_ref, r_ref, w1_ref, w2_ref, h_ref, hr_ref):
        a = p_ref[0] + p_ref[1]
        h = jnp.maximum(
            jnp.dot(a, w1_ref[...], preferred_element_type=jnp.float32)
            + r_ref[...], 0.0)
        h_ref[...] = h
        hr_ref[...] = jnp.dot(h, w2_ref[...],
                              preferred_element_type=jnp.float32)

    return pl.pallas_call(
        body,
        grid=(N // _BLK,),
        in_specs=[
            pl.BlockSpec((NC, _BLK, H), lambda i: (0, i, 0)),
            pl.BlockSpec((_BLK, H), lambda i: (i, 0)),
            pl.BlockSpec((H, H), lambda i: (0, 0)),
            pl.BlockSpec((H, H), lambda i: (0, 0)),
        ],
        out_specs=[
            pl.BlockSpec((_BLK, H), lambda i: (i, 0)),
            pl.BlockSpec((_BLK, H), lambda i: (i, 0)),
        ],
        out_shape=[
            jax.ShapeDtypeStruct((N, H), jnp.float32),
            jax.ShapeDtypeStruct((N, H), jnp.float32),
        ],
    )(p, xroot1, W_rel1, W_rel2)


def _tc_pool(p, hroot2, batch3):
    nblk = N // _BLK

    def body(p_ref, h_ref, bt_ref, o_ref, acc, cnt):
        i = pl.program_id(0)
        h2 = p_ref[0] + p_ref[1] + h_ref[...]
        seg = bt_ref[0]
        gids = lax.broadcasted_iota(jnp.int32, (G, _BLK), 0)
        mask = (seg == gids).astype(jnp.float32)

        @pl.when(i == 0)
        def _():
            acc[...] = jnp.zeros_like(acc)
            cnt[...] = jnp.zeros_like(cnt)

        acc[...] += jnp.dot(mask, h2, preferred_element_type=jnp.float32)
        cnt[...] += jnp.broadcast_to(
            jnp.sum(mask, axis=1, keepdims=True), (G, H))

        @pl.when(i == nblk - 1)
        def _():
            o_ref[...] = jnp.maximum(
                acc[...] / jnp.maximum(cnt[...], 1.0), 0.0)

    return pl.pallas_call(
        body,
        grid=(nblk,),
        in_specs=[
            pl.BlockSpec((NC, _BLK, H), lambda i: (0, i, 0)),
            pl.BlockSpec((_BLK, H), lambda i: (i, 0)),
            pl.BlockSpec((1, 1, _BLK), lambda i: (i, 0, 0)),
        ],
        out_specs=pl.BlockSpec((G, H), lambda i: (0, 0)),
        out_shape=jax.ShapeDtypeStruct((G, H), jnp.float32),
        scratch_shapes=[
            pltpu.VMEM((G, H), jnp.float32),
            pltpu.VMEM((G, H), jnp.float32),
        ],
    )(p, hroot2, batch3)


def kernel(x, edge_index, batch, W_rel1, b_rel1, W_root1,
           W_rel2, b_rel2, W_root2):
    src_flat = edge_index[0]
    dst = edge_index[1].reshape(NW, NCHUNK, CHUNK)
    zeros = jnp.zeros((NPAD, H), jnp.float32)
    batch3 = batch.reshape(N // _BLK, 1, _BLK)

    xroot1 = _tc_root(x, W_root1, b_rel1.reshape(1, H))
    p1 = _sc_aggregate(x, src_flat, dst, zeros)
    hmid, hrel2 = _tc_mid(p1, xroot1, W_rel1, W_rel2)
    hroot2 = _tc_root(hmid, W_root2, b_rel2.reshape(1, H))
    p2 = _sc_aggregate(hrel2, src_flat, dst, zeros)
    return _tc_pool(p2, hroot2, batch3)


# speedup vs baseline: 1.0135x; 1.0135x over previous
import functools

import jax
import jax.numpy as jnp
from jax import lax
from jax.experimental import pallas as pl
from jax.experimental.pallas import tpu as pltpu
from jax.experimental.pallas import tpu_sc as plsc

N = 10000
E = 320000
H = 128
G = 64

NC = 2
NS = 16
NW = NC * NS
EPW = E // NW
CHUNK = 80
NCHUNK = EPW // CHUNK
NPAD = 10240
RPT = NPAD // NS


def _sc_aggregate(x, src_flat, dst, zeros):
    mesh = plsc.VectorSubcoreMesh(core_axis_name="c", subcore_axis_name="s")

    @functools.partial(
        pl.kernel,
        out_type=jax.ShapeDtypeStruct((NC, NPAD, H), jnp.float32),
        mesh=mesh,
        scratch_types=[
            pltpu.VMEM((EPW,), jnp.int32),
            pltpu.VMEM((EPW,), jnp.int32),
            pltpu.VMEM((CHUNK, H), jnp.float32),
            pltpu.VMEM((CHUNK, H), jnp.float32),
            pltpu.VMEM_SHARED((NPAD, H), jnp.float32),
            pltpu.SemaphoreType.DMA,
            pltpu.SemaphoreType.DMA,
        ],
    )
    def agg(x_hbm, src_hbm, dst_hbm, z_hbm, out_hbm,
            src_v, dst_v, rows_a, rows_b, acc_sh, sem_a, sem_b):
        c = lax.axis_index("c")
        s = lax.axis_index("s")
        wid = c * NS + s
        pltpu.sync_copy(src_hbm.at[pl.ds(wid * EPW, EPW)], src_v)
        pltpu.sync_copy(dst_hbm.at[pl.ds(wid * EPW, EPW)], dst_v)
        pltpu.sync_copy(z_hbm.at[pl.ds(s * RPT, RPT)],
                        acc_sh.at[pl.ds(s * RPT, RPT)])
        plsc.subcore_barrier()

        def gather(j, rows, sem):
            pltpu.async_copy(
                x_hbm.at[src_v.at[pl.ds(j * CHUNK, CHUNK)]], rows, sem)

        def scat(j, rows, sem):
            pltpu.make_async_copy(
                x_hbm.at[src_v.at[pl.ds(j * CHUNK, CHUNK)]],
                rows, sem).wait()
            pltpu.sync_copy(
                rows, acc_sh.at[dst_v.at[pl.ds(j * CHUNK, CHUNK)]], add=True)

        gather(0, rows_a, sem_a)

        def pair(j, issue_next):
            gather(j + 1, rows_b, sem_b)
            scat(j, rows_a, sem_a)
            if issue_next:
                gather(j + 2, rows_a, sem_a)
            scat(j + 1, rows_b, sem_b)

        def body(p, carry):
            pair(2 * p, True)
            return carry

        lax.fori_loop(0, NCHUNK // 2, body, 0)
        scat(NCHUNK - 1, rows_a, sem_a)
        plsc.subcore_barrier()
        pltpu.sync_copy(acc_sh.at[pl.ds(s * RPT, RPT)],
                        out_hbm.at[c, pl.ds(s * RPT, RPT)])

    return agg(x, src_flat, dst, zeros)


_BLK = 1000


def _tc_root(x, W, b):
    def body(x_ref, w_ref, b_ref, o_ref):
        o_ref[...] = (jnp.dot(x_ref[...], w_ref[...],
                              preferred_element_type=jnp.float32)
                      + b_ref[...])

    return pl.pallas_call(
        body,
        grid=(N // _BLK,),
        in_specs=[
            pl.BlockSpec((_BLK, H), lambda i: (i, 0)),
            pl.BlockSpec((H, H), lambda i: (0, 0)),
            pl.BlockSpec((1, H), lambda i: (0, 0)),
        ],
        out_specs=pl.BlockSpec((_BLK, H), lambda i: (i, 0)),
        out_shape=jax.ShapeDtypeStruct((N, H), jnp.float32),
    )(x, W, b)


def _tc_mid(p, xroot1, W_rel1, W_rel2):
    def body(p_ref, r_ref, w1_ref, w2_ref, h_ref, hr_ref):
        a = p_ref[0] + p_ref[1]
        h = jnp.maximum(
            jnp.dot(a, w1_ref[...], preferred_element_type=jnp.float32)
            + r_ref[...], 0.0)
        h_ref[...] = h
        hr_ref[...] = jnp.dot(h, w2_ref[...],
                              preferred_element_type=jnp.float32)

    return pl.pallas_call(
        body,
        grid=(N // _BLK,),
        in_specs=[
            pl.BlockSpec((NC, _BLK, H), lambda i: (0, i, 0)),
            pl.BlockSpec((_BLK, H), lambda i: (i, 0)),
            pl.BlockSpec((H, H), lambda i: (0, 0)),
            pl.BlockSpec((H, H), lambda i: (0, 0)),
        ],
        out_specs=[
            pl.BlockSpec((_BLK, H), lambda i: (i, 0)),
            pl.BlockSpec((_BLK, H), lambda i: (i, 0)),
        ],
        out_shape=[
            jax.ShapeDtypeStruct((N, H), jnp.float32),
            jax.ShapeDtypeStruct((N, H), jnp.float32),
        ],
    )(p, xroot1, W_rel1, W_rel2)


def _tc_pool(p, hroot2, batch3):
    nblk = N // _BLK

    def body(p_ref, h_ref, bt_ref, o_ref, acc, cnt):
        i = pl.program_id(0)
        h2 = p_ref[0] + p_ref[1] + h_ref[...]
        seg = bt_ref[0]
        gids = lax.broadcasted_iota(jnp.int32, (G, _BLK), 0)
        mask = (seg == gids).astype(jnp.float32)

        @pl.when(i == 0)
        def _():
            acc[...] = jnp.zeros_like(acc)
            cnt[...] = jnp.zeros_like(cnt)

        acc[...] += jnp.dot(mask, h2, preferred_element_type=jnp.float32)
        cnt[...] += jnp.broadcast_to(
            jnp.sum(mask, axis=1, keepdims=True), (G, H))

        @pl.when(i == nblk - 1)
        def _():
            o_ref[...] = jnp.maximum(
                acc[...] / jnp.maximum(cnt[...], 1.0), 0.0)

    return pl.pallas_call(
        body,
        grid=(nblk,),
        in_specs=[
            pl.BlockSpec((NC, _BLK, H), lambda i: (0, i, 0)),
            pl.BlockSpec((_BLK, H), lambda i: (i, 0)),
            pl.BlockSpec((1, 1, _BLK), lambda i: (i, 0, 0)),
        ],
        out_specs=pl.BlockSpec((G, H), lambda i: (0, 0)),
        out_shape=jax.ShapeDtypeStruct((G, H), jnp.float32),
        scratch_shapes=[
            pltpu.VMEM((G, H), jnp.float32),
            pltpu.VMEM((G, H), jnp.float32),
        ],
    )(p, hroot2, batch3)


def kernel(x, edge_index, batch, W_rel1, b_rel1, W_root1,
           W_rel2, b_rel2, W_root2):
    src_flat = edge_index[0]
    dst = edge_index[1]
    zeros = jnp.zeros((NPAD, H), jnp.float32)
    batch3 = batch.reshape(N // _BLK, 1, _BLK)

    xroot1 = _tc_root(x, W_root1, b_rel1.reshape(1, H))
    p1 = _sc_aggregate(x, src_flat, dst, zeros)
    hmid, hrel2 = _tc_mid(p1, xroot1, W_rel1, W_rel2)
    hroot2 = _tc_root(hmid, W_root2, b_rel2.reshape(1, H))
    p2 = _sc_aggregate(hrel2, src_flat, dst, zeros)
    return _tc_pool(p2, hroot2, batch3)


# speedup vs baseline: 1.0507x; 1.0367x over previous
import functools

import jax
import jax.numpy as jnp
from jax import lax
from jax.experimental import pallas as pl
from jax.experimental.pallas import tpu as pltpu
from jax.experimental.pallas import tpu_sc as plsc

N = 10000
E = 320000
H = 128
G = 64

NC = 2
NS = 16
NW = NC * NS
EPW = E // NW
CHUNK = 96
NCHUNK = EPW // CHUNK
TAIL = EPW - NCHUNK * CHUNK
NPAD = 10240
RPT = NPAD // NS


def _sc_aggregate(x, src_flat, dst, zeros):
    mesh = plsc.VectorSubcoreMesh(core_axis_name="c", subcore_axis_name="s")

    @functools.partial(
        pl.kernel,
        out_type=jax.ShapeDtypeStruct((NC, NPAD, H), jnp.float32),
        mesh=mesh,
        scratch_types=[
            pltpu.VMEM((EPW,), jnp.int32),
            pltpu.VMEM((EPW,), jnp.int32),
            pltpu.VMEM((CHUNK, H), jnp.float32),
            pltpu.VMEM((CHUNK, H), jnp.float32),
            pltpu.VMEM_SHARED((NPAD, H), jnp.float32),
            pltpu.SemaphoreType.DMA,
            pltpu.SemaphoreType.DMA,
        ],
    )
    def agg(x_hbm, src_hbm, dst_hbm, z_hbm, out_hbm,
            src_v, dst_v, rows_a, rows_b, acc_sh, sem_a, sem_b):
        c = lax.axis_index("c")
        s = lax.axis_index("s")
        wid = c * NS + s
        pltpu.sync_copy(src_hbm.at[pl.ds(wid * EPW, EPW)], src_v)
        pltpu.sync_copy(dst_hbm.at[pl.ds(wid * EPW, EPW)], dst_v)
        pltpu.sync_copy(z_hbm.at[pl.ds(s * RPT, RPT)],
                        acc_sh.at[pl.ds(s * RPT, RPT)])
        plsc.subcore_barrier()

        def gather(j, rows, sem):
            pltpu.async_copy(
                x_hbm.at[src_v.at[pl.ds(j * CHUNK, CHUNK)]], rows, sem)

        def scat(j, rows, sem):
            pltpu.make_async_copy(
                x_hbm.at[src_v.at[pl.ds(j * CHUNK, CHUNK)]],
                rows, sem).wait()
            pltpu.sync_copy(
                rows, acc_sh.at[dst_v.at[pl.ds(j * CHUNK, CHUNK)]], add=True)

        gather(0, rows_a, sem_a)

        def pair(j, issue_next):
            gather(j + 1, rows_b, sem_b)
            scat(j, rows_a, sem_a)
            if issue_next:
                gather(j + 2, rows_a, sem_a)
            scat(j + 1, rows_b, sem_b)

        def body(p, carry):
            pair(2 * p, True)
            return carry

        assert NCHUNK % 2 == 0 and TAIL % 8 == 0
        lax.fori_loop(0, NCHUNK // 2 - 1, body, 0)
        pair(NCHUNK - 2, False)
        if TAIL:
            base = NCHUNK * CHUNK
            pltpu.async_copy(
                x_hbm.at[src_v.at[pl.ds(base, TAIL)]],
                rows_a.at[pl.ds(0, TAIL)], sem_a)
            pltpu.make_async_copy(
                x_hbm.at[src_v.at[pl.ds(base, TAIL)]],
                rows_a.at[pl.ds(0, TAIL)], sem_a).wait()
            pltpu.sync_copy(rows_a.at[pl.ds(0, TAIL)],
                            acc_sh.at[dst_v.at[pl.ds(base, TAIL)]], add=True)
        plsc.subcore_barrier()
        pltpu.sync_copy(acc_sh.at[pl.ds(s * RPT, RPT)],
                        out_hbm.at[c, pl.ds(s * RPT, RPT)])

    return agg(x, src_flat, dst, zeros)


_BLK = 1000


def _tc_root(x, W, b):
    def body(x_ref, w_ref, b_ref, o_ref):
        o_ref[...] = (jnp.dot(x_ref[...], w_ref[...],
                              preferred_element_type=jnp.float32)
                      + b_ref[...])

    return pl.pallas_call(
        body,
        grid=(N // _BLK,),
        in_specs=[
            pl.BlockSpec((_BLK, H), lambda i: (i, 0)),
            pl.BlockSpec((H, H), lambda i: (0, 0)),
            pl.BlockSpec((1, H), lambda i: (0, 0)),
        ],
        out_specs=pl.BlockSpec((_BLK, H), lambda i: (i, 0)),
        out_shape=jax.ShapeDtypeStruct((N, H), jnp.float32),
    )(x, W, b)


def _tc_mid(p, xroot1, W_rel1, W_rel2):
    def body(p_ref, r_ref, w1_ref, w2_ref, h_ref, hr_ref):
        a = p_ref[0] + p_ref[1]
        h = jnp.maximum(
            jnp.dot(a, w1_ref[...], preferred_element_type=jnp.float32)
            + r_ref[...], 0.0)
        h_ref[...] = h
        hr_ref[...] = jnp.dot(h, w2_ref[...],
                              preferred_element_type=jnp.float32)

    return pl.pallas_call(
        body,
        grid=(N // _BLK,),
        in_specs=[
            pl.BlockSpec((NC, _BLK, H), lambda i: (0, i, 0)),
            pl.BlockSpec((_BLK, H), lambda i: (i, 0)),
            pl.BlockSpec((H, H), lambda i: (0, 0)),
            pl.BlockSpec((H, H), lambda i: (0, 0)),
        ],
        out_specs=[
            pl.BlockSpec((_BLK, H), lambda i: (i, 0)),
            pl.BlockSpec((_BLK, H), lambda i: (i, 0)),
        ],
        out_shape=[
            jax.ShapeDtypeStruct((N, H), jnp.float32),
            jax.ShapeDtypeStruct((N, H), jnp.float32),
        ],
    )(p, xroot1, W_rel1, W_rel2)


def _tc_pool(p, hroot2, batch3):
    nblk = N // _BLK

    def body(p_ref, h_ref, bt_ref, o_ref, acc, cnt):
        i = pl.program_id(0)
        h2 = p_ref[0] + p_ref[1] + h_ref[...]
        seg = bt_ref[0]
        gids = lax.broadcasted_iota(jnp.int32, (G, _BLK), 0)
        mask = (seg == gids).astype(jnp.float32)

        @pl.when(i == 0)
        def _():
            acc[...] = jnp.zeros_like(acc)
            cnt[...] = jnp.zeros_like(cnt)

        acc[...] += jnp.dot(mask, h2, preferred_element_type=jnp.float32)
        cnt[...] += jnp.broadcast_to(
            jnp.sum(mask, axis=1, keepdims=True), (G, H))

        @pl.when(i == nblk - 1)
        def _():
            o_ref[...] = jnp.maximum(
                acc[...] / jnp.maximum(cnt[...], 1.0), 0.0)

    return pl.pallas_call(
        body,
        grid=(nblk,),
        in_specs=[
            pl.BlockSpec((NC, _BLK, H), lambda i: (0, i, 0)),
            pl.BlockSpec((_BLK, H), lambda i: (i, 0)),
            pl.BlockSpec((1, 1, _BLK), lambda i: (i, 0, 0)),
        ],
        out_specs=pl.BlockSpec((G, H), lambda i: (0, 0)),
        out_shape=jax.ShapeDtypeStruct((G, H), jnp.float32),
        scratch_shapes=[
            pltpu.VMEM((G, H), jnp.float32),
            pltpu.VMEM((G, H), jnp.float32),
        ],
    )(p, hroot2, batch3)


def kernel(x, edge_index, batch, W_rel1, b_rel1, W_root1,
           W_rel2, b_rel2, W_root2):
    src_flat = edge_index[0]
    dst = edge_index[1]
    zeros = jnp.zeros((NPAD, H), jnp.float32)
    batch3 = batch.reshape(N // _BLK, 1, _BLK)

    xroot1 = _tc_root(x, W_root1, b_rel1.reshape(1, H))
    p1 = _sc_aggregate(x, src_flat, dst, zeros)
    hmid, hrel2 = _tc_mid(p1, xroot1, W_rel1, W_rel2)
    hroot2 = _tc_root(hmid, W_root2, b_rel2.reshape(1, H))
    p2 = _sc_aggregate(hrel2, src_flat, dst, zeros)
    return _tc_pool(p2, hroot2, batch3)


# speedup vs baseline: 1.1509x; 1.0953x over previous
import functools

import jax
import jax.numpy as jnp
from jax import lax
from jax.experimental import pallas as pl
from jax.experimental.pallas import tpu as pltpu
from jax.experimental.pallas import tpu_sc as plsc

N = 10000
E = 320000
H = 128
G = 64

NC = 2
NS = 16
NW = NC * NS
EPW = E // NW
CHUNK = 64
NCHUNK = EPW // CHUNK
TAIL = EPW - NCHUNK * CHUNK
NPAD = 10240
RPT = NPAD // NS


def _sc_aggregate(x, src_flat, dst, zeros):
    mesh = plsc.VectorSubcoreMesh(core_axis_name="c", subcore_axis_name="s")

    @functools.partial(
        pl.kernel,
        out_type=jax.ShapeDtypeStruct((NC, NPAD, H), jnp.float32),
        mesh=mesh,
        scratch_types=[
            pltpu.VMEM((EPW,), jnp.int32),
            pltpu.VMEM((EPW,), jnp.int32),
            pltpu.VMEM((CHUNK, H), jnp.float32),
            pltpu.VMEM((CHUNK, H), jnp.float32),
            pltpu.VMEM((CHUNK, H), jnp.float32),
            pltpu.VMEM_SHARED((NPAD, H), jnp.float32),
            pltpu.SemaphoreType.DMA,
            pltpu.SemaphoreType.DMA,
            pltpu.SemaphoreType.DMA,
        ],
    )
    def agg(x_hbm, src_hbm, dst_hbm, z_hbm, out_hbm,
            src_v, dst_v, rows_a, rows_b, rows_c, acc_sh,
            sem_a, sem_b, sem_c):
        c = lax.axis_index("c")
        s = lax.axis_index("s")
        wid = c * NS + s
        pltpu.sync_copy(src_hbm.at[pl.ds(wid * EPW, EPW)], src_v)
        pltpu.sync_copy(dst_hbm.at[pl.ds(wid * EPW, EPW)], dst_v)
        pltpu.sync_copy(z_hbm.at[pl.ds(s * RPT, RPT)],
                        acc_sh.at[pl.ds(s * RPT, RPT)])
        plsc.subcore_barrier()

        def gather(j, rows, sem):
            pltpu.async_copy(
                x_hbm.at[src_v.at[pl.ds(j * CHUNK, CHUNK)]], rows, sem)

        def scat(j, rows, sem):
            pltpu.make_async_copy(
                x_hbm.at[src_v.at[pl.ds(j * CHUNK, CHUNK)]],
                rows, sem).wait()
            pltpu.sync_copy(
                rows, acc_sh.at[dst_v.at[pl.ds(j * CHUNK, CHUNK)]], add=True)

        gather(0, rows_a, sem_a)
        gather(1, rows_b, sem_b)

        def triple(j, issue_next):
            gather(j + 2, rows_c, sem_c)
            scat(j, rows_a, sem_a)
            if issue_next:
                gather(j + 3, rows_a, sem_a)
            scat(j + 1, rows_b, sem_b)
            if issue_next:
                gather(j + 4, rows_b, sem_b)
            scat(j + 2, rows_c, sem_c)

        def body(p, carry):
            triple(3 * p, True)
            return carry

        assert NCHUNK % 3 == 0 and TAIL % 8 == 0
        lax.fori_loop(0, NCHUNK // 3 - 1, body, 0)
        triple(NCHUNK - 3, False)
        if TAIL:
            base = NCHUNK * CHUNK
            pltpu.async_copy(
                x_hbm.at[src_v.at[pl.ds(base, TAIL)]],
                rows_a.at[pl.ds(0, TAIL)], sem_a)
            pltpu.make_async_copy(
                x_hbm.at[src_v.at[pl.ds(base, TAIL)]],
                rows_a.at[pl.ds(0, TAIL)], sem_a).wait()
            pltpu.sync_copy(rows_a.at[pl.ds(0, TAIL)],
                            acc_sh.at[dst_v.at[pl.ds(base, TAIL)]], add=True)
        plsc.subcore_barrier()
        pltpu.sync_copy(acc_sh.at[pl.ds(s * RPT, RPT)],
                        out_hbm.at[c, pl.ds(s * RPT, RPT)])

    return agg(x, src_flat, dst, zeros)


_BLK = 1000


def _tc_root(x, W, b):
    def body(x_ref, w_ref, b_ref, o_ref):
        o_ref[...] = (jnp.dot(x_ref[...], w_ref[...],
                              preferred_element_type=jnp.float32)
                      + b_ref[...])

    return pl.pallas_call(
        body,
        grid=(N // _BLK,),
        in_specs=[
            pl.BlockSpec((_BLK, H), lambda i: (i, 0)),
            pl.BlockSpec((H, H), lambda i: (0, 0)),
            pl.BlockSpec((1, H), lambda i: (0, 0)),
        ],
        out_specs=pl.BlockSpec((_BLK, H), lambda i: (i, 0)),
        out_shape=jax.ShapeDtypeStruct((N, H), jnp.float32),
    )(x, W, b)


def _tc_mid(p, xroot1, W_rel1, W_rel2):
    def body(p_ref, r_ref, w1_ref, w2_ref, h_ref, hr_ref):
        a = p_ref[0] + p_ref[1]
        h = jnp.maximum(
            jnp.dot(a, w1_ref[...], preferred_element_type=jnp.float32)
            + r_ref[...], 0.0)
        h_ref[...] = h
        hr_ref[...] = jnp.dot(h, w2_ref[...],
                              preferred_element_type=jnp.float32)

    return pl.pallas_call(
        body,
        grid=(N // _BLK,),
        in_specs=[
            pl.BlockSpec((NC, _BLK, H), lambda i: (0, i, 0)),
            pl.BlockSpec((_BLK, H), lambda i: (i, 0)),
            pl.BlockSpec((H, H), lambda i: (0, 0)),
            pl.BlockSpec((H, H), lambda i: (0, 0)),
        ],
        out_specs=[
            pl.BlockSpec((_BLK, H), lambda i: (i, 0)),
            pl.BlockSpec((_BLK, H), lambda i: (i, 0)),
        ],
        out_shape=[
            jax.ShapeDtypeStruct((N, H), jnp.float32),
            jax.ShapeDtypeStruct((N, H), jnp.float32),
        ],
    )(p, xroot1, W_rel1, W_rel2)


def _tc_pool(p, hroot2, batch3):
    nblk = N // _BLK

    def body(p_ref, h_ref, bt_ref, o_ref, acc, cnt):
        i = pl.program_id(0)
        h2 = p_ref[0] + p_ref[1] + h_ref[...]
        seg = bt_ref[0]
        gids = lax.broadcasted_iota(jnp.int32, (G, _BLK), 0)
        mask = (seg == gids).astype(jnp.float32)

        @pl.when(i == 0)
        def _():
            acc[...] = jnp.zeros_like(acc)
            cnt[...] = jnp.zeros_like(cnt)

        acc[...] += jnp.dot(mask, h2, preferred_element_type=jnp.float32)
        cnt[...] += jnp.broadcast_to(
            jnp.sum(mask, axis=1, keepdims=True), (G, H))

        @pl.when(i == nblk - 1)
        def _():
            o_ref[...] = jnp.maximum(
                acc[...] / jnp.maximum(cnt[...], 1.0), 0.0)

    return pl.pallas_call(
        body,
        grid=(nblk,),
        in_specs=[
            pl.BlockSpec((NC, _BLK, H), lambda i: (0, i, 0)),
            pl.BlockSpec((_BLK, H), lambda i: (i, 0)),
            pl.BlockSpec((1, 1, _BLK), lambda i: (i, 0, 0)),
        ],
        out_specs=pl.BlockSpec((G, H), lambda i: (0, 0)),
        out_shape=jax.ShapeDtypeStruct((G, H), jnp.float32),
        scratch_shapes=[
            pltpu.VMEM((G, H), jnp.float32),
            pltpu.VMEM((G, H), jnp.float32),
        ],
    )(p, hroot2, batch3)


def kernel(x, edge_index, batch, W_rel1, b_rel1, W_root1,
           W_rel2, b_rel2, W_root2):
    src_flat = edge_index[0]
    dst = edge_index[1]
    zeros = jnp.zeros((NPAD, H), jnp.float32)
    batch3 = batch.reshape(N // _BLK, 1, _BLK)

    xroot1 = _tc_root(x, W_root1, b_rel1.reshape(1, H))
    p1 = _sc_aggregate(x, src_flat, dst, zeros)
    hmid, hrel2 = _tc_mid(p1, xroot1, W_rel1, W_rel2)
    hroot2 = _tc_root(hmid, W_root2, b_rel2.reshape(1, H))
    p2 = _sc_aggregate(hrel2, src_flat, dst, zeros)
    return _tc_pool(p2, hroot2, batch3)


# speedup vs baseline: 1.1992x; 1.0420x over previous
import functools

import jax
import jax.numpy as jnp
from jax import lax
from jax.experimental import pallas as pl
from jax.experimental.pallas import tpu as pltpu
from jax.experimental.pallas import tpu_sc as plsc

N = 10000
E = 320000
H = 128
G = 64

NC = 2
NS = 16
NW = NC * NS
EPW = E // NW
CHUNK = 48
NCHUNK = EPW // CHUNK
TAIL = EPW - NCHUNK * CHUNK
NPAD = 10240
RPT = NPAD // NS


def _sc_aggregate(x, src_flat, dst, zeros):
    mesh = plsc.VectorSubcoreMesh(core_axis_name="c", subcore_axis_name="s")

    @functools.partial(
        pl.kernel,
        out_type=jax.ShapeDtypeStruct((NC, NPAD, H), jnp.float32),
        mesh=mesh,
        scratch_types=[
            pltpu.VMEM((EPW,), jnp.int32),
            pltpu.VMEM((EPW,), jnp.int32),
            pltpu.VMEM((CHUNK, H), jnp.float32),
            pltpu.VMEM((CHUNK, H), jnp.float32),
            pltpu.VMEM((CHUNK, H), jnp.float32),
            pltpu.VMEM((CHUNK, H), jnp.float32),
            pltpu.VMEM_SHARED((NPAD, H), jnp.float32),
            pltpu.SemaphoreType.DMA,
            pltpu.SemaphoreType.DMA,
            pltpu.SemaphoreType.DMA,
            pltpu.SemaphoreType.DMA,
        ],
    )
    def agg(x_hbm, src_hbm, dst_hbm, z_hbm, out_hbm,
            src_v, dst_v, rows_a, rows_b, rows_c, rows_d, acc_sh,
            sem_a, sem_b, sem_c, sem_d):
        c = lax.axis_index("c")
        s = lax.axis_index("s")
        wid = c * NS + s
        pltpu.sync_copy(src_hbm.at[pl.ds(wid * EPW, EPW)], src_v)
        pltpu.sync_copy(dst_hbm.at[pl.ds(wid * EPW, EPW)], dst_v)
        pltpu.sync_copy(z_hbm.at[pl.ds(s * RPT, RPT)],
                        acc_sh.at[pl.ds(s * RPT, RPT)])
        plsc.subcore_barrier()

        def gather(j, rows, sem):
            pltpu.async_copy(
                x_hbm.at[src_v.at[pl.ds(j * CHUNK, CHUNK)]], rows, sem)

        def scat(j, rows, sem):
            pltpu.make_async_copy(
                x_hbm.at[src_v.at[pl.ds(j * CHUNK, CHUNK)]],
                rows, sem).wait()
            pltpu.sync_copy(
                rows, acc_sh.at[dst_v.at[pl.ds(j * CHUNK, CHUNK)]], add=True)

        gather(0, rows_a, sem_a)
        gather(1, rows_b, sem_b)
        gather(2, rows_c, sem_c)

        def quad(j, issue_next):
            gather(j + 3, rows_d, sem_d)
            scat(j, rows_a, sem_a)
            if issue_next:
                gather(j + 4, rows_a, sem_a)
            scat(j + 1, rows_b, sem_b)
            if issue_next:
                gather(j + 5, rows_b, sem_b)
            scat(j + 2, rows_c, sem_c)
            if issue_next:
                gather(j + 6, rows_c, sem_c)
            scat(j + 3, rows_d, sem_d)

        def body(p, carry):
            quad(4 * p, True)
            return carry

        assert NCHUNK % 4 == 0 and TAIL % 8 == 0
        lax.fori_loop(0, NCHUNK // 4 - 1, body, 0)
        quad(NCHUNK - 4, False)
        if TAIL:
            base = NCHUNK * CHUNK
            pltpu.async_copy(
                x_hbm.at[src_v.at[pl.ds(base, TAIL)]],
                rows_a.at[pl.ds(0, TAIL)], sem_a)
            pltpu.make_async_copy(
                x_hbm.at[src_v.at[pl.ds(base, TAIL)]],
                rows_a.at[pl.ds(0, TAIL)], sem_a).wait()
            pltpu.sync_copy(rows_a.at[pl.ds(0, TAIL)],
                            acc_sh.at[dst_v.at[pl.ds(base, TAIL)]], add=True)
        plsc.subcore_barrier()
        pltpu.sync_copy(acc_sh.at[pl.ds(s * RPT, RPT)],
                        out_hbm.at[c, pl.ds(s * RPT, RPT)])

    return agg(x, src_flat, dst, zeros)


_BLK = 1000


def _tc_root(x, W, b):
    def body(x_ref, w_ref, b_ref, o_ref):
        o_ref[...] = (jnp.dot(x_ref[...], w_ref[...],
                              preferred_element_type=jnp.float32)
                      + b_ref[...])

    return pl.pallas_call(
        body,
        grid=(N // _BLK,),
        in_specs=[
            pl.BlockSpec((_BLK, H), lambda i: (i, 0)),
            pl.BlockSpec((H, H), lambda i: (0, 0)),
            pl.BlockSpec((1, H), lambda i: (0, 0)),
        ],
        out_specs=pl.BlockSpec((_BLK, H), lambda i: (i, 0)),
        out_shape=jax.ShapeDtypeStruct((N, H), jnp.float32),
    )(x, W, b)


def _tc_mid(p, xroot1, W_rel1, W_rel2):
    def body(p_ref, r_ref, w1_ref, w2_ref, h_ref, hr_ref):
        a = p_ref[0] + p_ref[1]
        h = jnp.maximum(
            jnp.dot(a, w1_ref[...], preferred_element_type=jnp.float32)
            + r_ref[...], 0.0)
        h_ref[...] = h
        hr_ref[...] = jnp.dot(h, w2_ref[...],
                              preferred_element_type=jnp.float32)

    return pl.pallas_call(
        body,
        grid=(N // _BLK,),
        in_specs=[
            pl.BlockSpec((NC, _BLK, H), lambda i: (0, i, 0)),
            pl.BlockSpec((_BLK, H), lambda i: (i, 0)),
            pl.BlockSpec((H, H), lambda i: (0, 0)),
            pl.BlockSpec((H, H), lambda i: (0, 0)),
        ],
        out_specs=[
            pl.BlockSpec((_BLK, H), lambda i: (i, 0)),
            pl.BlockSpec((_BLK, H), lambda i: (i, 0)),
        ],
        out_shape=[
            jax.ShapeDtypeStruct((N, H), jnp.float32),
            jax.ShapeDtypeStruct((N, H), jnp.float32),
        ],
    )(p, xroot1, W_rel1, W_rel2)


def _tc_pool(p, hroot2, batch3):
    nblk = N // _BLK

    def body(p_ref, h_ref, bt_ref, o_ref, acc, cnt):
        i = pl.program_id(0)
        h2 = p_ref[0] + p_ref[1] + h_ref[...]
        seg = bt_ref[0]
        gids = lax.broadcasted_iota(jnp.int32, (G, _BLK), 0)
        mask = (seg == gids).astype(jnp.float32)

        @pl.when(i == 0)
        def _():
            acc[...] = jnp.zeros_like(acc)
            cnt[...] = jnp.zeros_like(cnt)

        acc[...] += jnp.dot(mask, h2, preferred_element_type=jnp.float32)
        cnt[...] += jnp.broadcast_to(
            jnp.sum(mask, axis=1, keepdims=True), (G, H))

        @pl.when(i == nblk - 1)
        def _():
            o_ref[...] = jnp.maximum(
                acc[...] / jnp.maximum(cnt[...], 1.0), 0.0)

    return pl.pallas_call(
        body,
        grid=(nblk,),
        in_specs=[
            pl.BlockSpec((NC, _BLK, H), lambda i: (0, i, 0)),
            pl.BlockSpec((_BLK, H), lambda i: (i, 0)),
            pl.BlockSpec((1, 1, _BLK), lambda i: (i, 0, 0)),
        ],
        out_specs=pl.BlockSpec((G, H), lambda i: (0, 0)),
        out_shape=jax.ShapeDtypeStruct((G, H), jnp.float32),
        scratch_shapes=[
            pltpu.VMEM((G, H), jnp.float32),
            pltpu.VMEM((G, H), jnp.float32),
        ],
    )(p, hroot2, batch3)


def kernel(x, edge_index, batch, W_rel1, b_rel1, W_root1,
           W_rel2, b_rel2, W_root2):
    src_flat = edge_index[0]
    dst = edge_index[1]
    zeros = jnp.zeros((NPAD, H), jnp.float32)
    batch3 = batch.reshape(N // _BLK, 1, _BLK)

    xroot1 = _tc_root(x, W_root1, b_rel1.reshape(1, H))
    p1 = _sc_aggregate(x, src_flat, dst, zeros)
    hmid, hrel2 = _tc_mid(p1, xroot1, W_rel1, W_rel2)
    hroot2 = _tc_root(hmid, W_root2, b_rel2.reshape(1, H))
    p2 = _sc_aggregate(hrel2, src_flat, dst, zeros)
    return _tc_pool(p2, hroot2, batch3)


# speedup vs baseline: 1.2215x; 1.0185x over previous
import functools

import jax
import jax.numpy as jnp
from jax import lax
from jax.experimental import pallas as pl
from jax.experimental.pallas import tpu as pltpu
from jax.experimental.pallas import tpu_sc as plsc

N = 10000
E = 320000
H = 128
G = 64

NC = 2
NS = 16
NW = NC * NS
EPW = E // NW
CHUNK = 40
NCHUNK = EPW // CHUNK
TAIL = EPW - NCHUNK * CHUNK
NPAD = 10240
RPT = NPAD // NS


def _sc_aggregate(x, src_flat, dst, zeros):
    mesh = plsc.VectorSubcoreMesh(core_axis_name="c", subcore_axis_name="s")

    @functools.partial(
        pl.kernel,
        out_type=jax.ShapeDtypeStruct((NC, NPAD, H), jnp.float32),
        mesh=mesh,
        scratch_types=[
            pltpu.VMEM((EPW,), jnp.int32),
            pltpu.VMEM((EPW,), jnp.int32),
            pltpu.VMEM((CHUNK, H), jnp.float32),
            pltpu.VMEM((CHUNK, H), jnp.float32),
            pltpu.VMEM((CHUNK, H), jnp.float32),
            pltpu.VMEM((CHUNK, H), jnp.float32),
            pltpu.VMEM((CHUNK, H), jnp.float32),
            pltpu.VMEM_SHARED((NPAD, H), jnp.float32),
            pltpu.SemaphoreType.DMA,
            pltpu.SemaphoreType.DMA,
            pltpu.SemaphoreType.DMA,
            pltpu.SemaphoreType.DMA,
            pltpu.SemaphoreType.DMA,
        ],
    )
    def agg(x_hbm, src_hbm, dst_hbm, z_hbm, out_hbm,
            src_v, dst_v, rows_a, rows_b, rows_c, rows_d, rows_e, acc_sh,
            sem_a, sem_b, sem_c, sem_d, sem_e):
        c = lax.axis_index("c")
        s = lax.axis_index("s")
        wid = c * NS + s
        pltpu.sync_copy(src_hbm.at[pl.ds(wid * EPW, EPW)], src_v)
        pltpu.sync_copy(dst_hbm.at[pl.ds(wid * EPW, EPW)], dst_v)
        pltpu.sync_copy(z_hbm.at[pl.ds(s * RPT, RPT)],
                        acc_sh.at[pl.ds(s * RPT, RPT)])
        plsc.subcore_barrier()

        def gather(j, rows, sem):
            pltpu.async_copy(
                x_hbm.at[src_v.at[pl.ds(j * CHUNK, CHUNK)]], rows, sem)

        def scat(j, rows, sem):
            pltpu.make_async_copy(
                x_hbm.at[src_v.at[pl.ds(j * CHUNK, CHUNK)]],
                rows, sem).wait()
            pltpu.sync_copy(
                rows, acc_sh.at[dst_v.at[pl.ds(j * CHUNK, CHUNK)]], add=True)

        slots = [(rows_a, sem_a), (rows_b, sem_b), (rows_c, sem_c),
                 (rows_d, sem_d), (rows_e, sem_e)]
        depth = len(slots)
        for k in range(depth - 1):
            gather(k, *slots[k])

        def group(j, issue_next):
            gather(j + depth - 1, *slots[depth - 1])
            for k in range(depth - 1):
                scat(j + k, *slots[k])
                if issue_next:
                    gather(j + depth + k, *slots[k])
            scat(j + depth - 1, *slots[depth - 1])

        def body(p, carry):
            group(depth * p, True)
            return carry

        assert NCHUNK % depth == 0 and TAIL % 8 == 0
        lax.fori_loop(0, NCHUNK // depth - 1, body, 0)
        group(NCHUNK - depth, False)
        if TAIL:
            base = NCHUNK * CHUNK
            pltpu.async_copy(
                x_hbm.at[src_v.at[pl.ds(base, TAIL)]],
                rows_a.at[pl.ds(0, TAIL)], sem_a)
            pltpu.make_async_copy(
                x_hbm.at[src_v.at[pl.ds(base, TAIL)]],
                rows_a.at[pl.ds(0, TAIL)], sem_a).wait()
            pltpu.sync_copy(rows_a.at[pl.ds(0, TAIL)],
                            acc_sh.at[dst_v.at[pl.ds(base, TAIL)]], add=True)
        plsc.subcore_barrier()
        pltpu.sync_copy(acc_sh.at[pl.ds(s * RPT, RPT)],
                        out_hbm.at[c, pl.ds(s * RPT, RPT)])

    return agg(x, src_flat, dst, zeros)


_BLK = 1000


def _tc_root(x, W, b):
    def body(x_ref, w_ref, b_ref, o_ref):
        o_ref[...] = (jnp.dot(x_ref[...], w_ref[...],
                              preferred_element_type=jnp.float32)
                      + b_ref[...])

    return pl.pallas_call(
        body,
        grid=(N // _BLK,),
        in_specs=[
            pl.BlockSpec((_BLK, H), lambda i: (i, 0)),
            pl.BlockSpec((H, H), lambda i: (0, 0)),
            pl.BlockSpec((1, H), lambda i: (0, 0)),
        ],
        out_specs=pl.BlockSpec((_BLK, H), lambda i: (i, 0)),
        out_shape=jax.ShapeDtypeStruct((N, H), jnp.float32),
    )(x, W, b)


def _tc_mid(p, xroot1, W_rel1, W_rel2):
    def body(p_ref, r_ref, w1_ref, w2_ref, h_ref, hr_ref):
        a = p_ref[0] + p_ref[1]
        h = jnp.maximum(
            jnp.dot(a, w1_ref[...], preferred_element_type=jnp.float32)
            + r_ref[...], 0.0)
        h_ref[...] = h
        hr_ref[...] = jnp.dot(h, w2_ref[...],
                              preferred_element_type=jnp.float32)

    return pl.pallas_call(
        body,
        grid=(N // _BLK,),
        in_specs=[
            pl.BlockSpec((NC, _BLK, H), lambda i: (0, i, 0)),
            pl.BlockSpec((_BLK, H), lambda i: (i, 0)),
            pl.BlockSpec((H, H), lambda i: (0, 0)),
            pl.BlockSpec((H, H), lambda i: (0, 0)),
        ],
        out_specs=[
            pl.BlockSpec((_BLK, H), lambda i: (i, 0)),
            pl.BlockSpec((_BLK, H), lambda i: (i, 0)),
        ],
        out_shape=[
            jax.ShapeDtypeStruct((N, H), jnp.float32),
            jax.ShapeDtypeStruct((N, H), jnp.float32),
        ],
    )(p, xroot1, W_rel1, W_rel2)


def _tc_pool(p, hroot2, batch3):
    nblk = N // _BLK

    def body(p_ref, h_ref, bt_ref, o_ref, acc, cnt):
        i = pl.program_id(0)
        h2 = p_ref[0] + p_ref[1] + h_ref[...]
        seg = bt_ref[0]
        gids = lax.broadcasted_iota(jnp.int32, (G, _BLK), 0)
        mask = (seg == gids).astype(jnp.float32)

        @pl.when(i == 0)
        def _():
            acc[...] = jnp.zeros_like(acc)
            cnt[...] = jnp.zeros_like(cnt)

        acc[...] += jnp.dot(mask, h2, preferred_element_type=jnp.float32)
        cnt[...] += jnp.broadcast_to(
            jnp.sum(mask, axis=1, keepdims=True), (G, H))

        @pl.when(i == nblk - 1)
        def _():
            o_ref[...] = jnp.maximum(
                acc[...] / jnp.maximum(cnt[...], 1.0), 0.0)

    return pl.pallas_call(
        body,
        grid=(nblk,),
        in_specs=[
            pl.BlockSpec((NC, _BLK, H), lambda i: (0, i, 0)),
            pl.BlockSpec((_BLK, H), lambda i: (i, 0)),
            pl.BlockSpec((1, 1, _BLK), lambda i: (i, 0, 0)),
        ],
        out_specs=pl.BlockSpec((G, H), lambda i: (0, 0)),
        out_shape=jax.ShapeDtypeStruct((G, H), jnp.float32),
        scratch_shapes=[
            pltpu.VMEM((G, H), jnp.float32),
            pltpu.VMEM((G, H), jnp.float32),
        ],
    )(p, hroot2, batch3)


def kernel(x, edge_index, batch, W_rel1, b_rel1, W_root1,
           W_rel2, b_rel2, W_root2):
    src_flat = edge_index[0]
    dst = edge_index[1]
    zeros = jnp.zeros((NPAD, H), jnp.float32)
    batch3 = batch.reshape(N // _BLK, 1, _BLK)

    xroot1 = _tc_root(x, W_root1, b_rel1.reshape(1, H))
    p1 = _sc_aggregate(x, src_flat, dst, zeros)
    hmid, hrel2 = _tc_mid(p1, xroot1, W_rel1, W_rel2)
    hroot2 = _tc_root(hmid, W_root2, b_rel2.reshape(1, H))
    p2 = _sc_aggregate(hrel2, src_flat, dst, zeros)
    return _tc_pool(p2, hroot2, batch3)


# speedup vs baseline: 1.2523x; 1.0253x over previous
import functools

import jax
import jax.numpy as jnp
from jax import lax
from jax.experimental import pallas as pl
from jax.experimental.pallas import tpu as pltpu
from jax.experimental.pallas import tpu_sc as plsc

N = 10000
E = 320000
H = 128
G = 64

NC = 2
NS = 16
NW = NC * NS
EPW = E // NW
CHUNK = 40
NCHUNK = EPW // CHUNK
TAIL = EPW - NCHUNK * CHUNK
NPAD = 10240
RPT = NPAD // NS


def _sc_aggregate(x, src_flat, dst):
    mesh = plsc.VectorSubcoreMesh(core_axis_name="c", subcore_axis_name="s")

    @functools.partial(
        pl.kernel,
        out_type=jax.ShapeDtypeStruct((NC, NPAD, H), jnp.float32),
        mesh=mesh,
        scratch_types=[
            pltpu.VMEM((EPW,), jnp.int32),
            pltpu.VMEM((EPW,), jnp.int32),
            pltpu.VMEM((CHUNK, H), jnp.float32),
            pltpu.VMEM((CHUNK, H), jnp.float32),
            pltpu.VMEM((CHUNK, H), jnp.float32),
            pltpu.VMEM((CHUNK, H), jnp.float32),
            pltpu.VMEM((CHUNK, H), jnp.float32),
            pltpu.VMEM_SHARED((NPAD, H), jnp.float32),
            pltpu.SemaphoreType.DMA,
            pltpu.SemaphoreType.DMA,
            pltpu.SemaphoreType.DMA,
            pltpu.SemaphoreType.DMA,
            pltpu.SemaphoreType.DMA,
        ],
    )
    def agg(x_hbm, src_hbm, dst_hbm, out_hbm,
            src_v, dst_v, rows_a, rows_b, rows_c, rows_d, rows_e, acc_sh,
            sem_a, sem_b, sem_c, sem_d, sem_e):
        c = lax.axis_index("c")
        s = lax.axis_index("s")
        wid = c * NS + s
        pltpu.sync_copy(src_hbm.at[pl.ds(wid * EPW, EPW)], src_v)
        pltpu.sync_copy(dst_hbm.at[pl.ds(wid * EPW, EPW)], dst_v)
        zv = jnp.zeros((16,), jnp.float32)
        for i in range(CHUNK):
            for k in range(H // 16):
                rows_a[i, pl.ds(k * 16, 16)] = zv
        assert RPT % CHUNK == 0
        for r in range(RPT // CHUNK):
            pltpu.sync_copy(rows_a,
                            acc_sh.at[pl.ds(s * RPT + r * CHUNK, CHUNK)])
        plsc.subcore_barrier()

        def gather(j, rows, sem):
            pltpu.async_copy(
                x_hbm.at[src_v.at[pl.ds(j * CHUNK, CHUNK)]], rows, sem)

        def scat(j, rows, sem):
            pltpu.make_async_copy(
                x_hbm.at[src_v.at[pl.ds(j * CHUNK, CHUNK)]],
                rows, sem).wait()
            pltpu.sync_copy(
                rows, acc_sh.at[dst_v.at[pl.ds(j * CHUNK, CHUNK)]], add=True)

        slots = [(rows_a, sem_a), (rows_b, sem_b), (rows_c, sem_c),
                 (rows_d, sem_d), (rows_e, sem_e)]
        depth = len(slots)
        for k in range(depth - 1):
            gather(k, *slots[k])

        def group(j, issue_next):
            gather(j + depth - 1, *slots[depth - 1])
            for k in range(depth - 1):
                scat(j + k, *slots[k])
                if issue_next:
                    gather(j + depth + k, *slots[k])
            scat(j + depth - 1, *slots[depth - 1])

        def body(p, carry):
            group(depth * p, True)
            return carry

        assert NCHUNK % depth == 0 and TAIL % 8 == 0
        lax.fori_loop(0, NCHUNK // depth - 1, body, 0)
        group(NCHUNK - depth, False)
        if TAIL:
            base = NCHUNK * CHUNK
            pltpu.async_copy(
                x_hbm.at[src_v.at[pl.ds(base, TAIL)]],
                rows_a.at[pl.ds(0, TAIL)], sem_a)
            pltpu.make_async_copy(
                x_hbm.at[src_v.at[pl.ds(base, TAIL)]],
                rows_a.at[pl.ds(0, TAIL)], sem_a).wait()
            pltpu.sync_copy(rows_a.at[pl.ds(0, TAIL)],
                            acc_sh.at[dst_v.at[pl.ds(base, TAIL)]], add=True)
        plsc.subcore_barrier()
        pltpu.sync_copy(acc_sh.at[pl.ds(s * RPT, RPT)],
                        out_hbm.at[c, pl.ds(s * RPT, RPT)])

    return agg(x, src_flat, dst)


_BLK = 1000


def _tc_root(x, W, b):
    def body(x_ref, w_ref, b_ref, o_ref):
        o_ref[...] = (jnp.dot(x_ref[...], w_ref[...],
                              preferred_element_type=jnp.float32)
                      + b_ref[...])

    return pl.pallas_call(
        body,
        grid=(N // _BLK,),
        in_specs=[
            pl.BlockSpec((_BLK, H), lambda i: (i, 0)),
            pl.BlockSpec((H, H), lambda i: (0, 0)),
            pl.BlockSpec((1, H), lambda i: (0, 0)),
        ],
        out_specs=pl.BlockSpec((_BLK, H), lambda i: (i, 0)),
        out_shape=jax.ShapeDtypeStruct((N, H), jnp.float32),
    )(x, W, b)


def _tc_mid(p, xroot1, W_rel1, W_rel2):
    def body(p_ref, r_ref, w1_ref, w2_ref, h_ref, hr_ref):
        a = p_ref[0] + p_ref[1]
        h = jnp.maximum(
            jnp.dot(a, w1_ref[...], preferred_element_type=jnp.float32)
            + r_ref[...], 0.0)
        h_ref[...] = h
        hr_ref[...] = jnp.dot(h, w2_ref[...],
                              preferred_element_type=jnp.float32)

    return pl.pallas_call(
        body,
        grid=(N // _BLK,),
        in_specs=[
            pl.BlockSpec((NC, _BLK, H), lambda i: (0, i, 0)),
            pl.BlockSpec((_BLK, H), lambda i: (i, 0)),
            pl.BlockSpec((H, H), lambda i: (0, 0)),
            pl.BlockSpec((H, H), lambda i: (0, 0)),
        ],
        out_specs=[
            pl.BlockSpec((_BLK, H), lambda i: (i, 0)),
            pl.BlockSpec((_BLK, H), lambda i: (i, 0)),
        ],
        out_shape=[
            jax.ShapeDtypeStruct((N, H), jnp.float32),
            jax.ShapeDtypeStruct((N, H), jnp.float32),
        ],
    )(p, xroot1, W_rel1, W_rel2)


def _tc_pool(p, hroot2, batch3):
    nblk = N // _BLK

    def body(p_ref, h_ref, bt_ref, o_ref, acc, cnt):
        i = pl.program_id(0)
        h2 = p_ref[0] + p_ref[1] + h_ref[...]
        seg = bt_ref[0]
        gids = lax.broadcasted_iota(jnp.int32, (G, _BLK), 0)
        mask = (seg == gids).astype(jnp.float32)

        @pl.when(i == 0)
        def _():
            acc[...] = jnp.zeros_like(acc)
            cnt[...] = jnp.zeros_like(cnt)

        acc[...] += jnp.dot(mask, h2, preferred_element_type=jnp.float32)
        cnt[...] += jnp.broadcast_to(
            jnp.sum(mask, axis=1, keepdims=True), (G, H))

        @pl.when(i == nblk - 1)
        def _():
            o_ref[...] = jnp.maximum(
                acc[...] / jnp.maximum(cnt[...], 1.0), 0.0)

    return pl.pallas_call(
        body,
        grid=(nblk,),
        in_specs=[
            pl.BlockSpec((NC, _BLK, H), lambda i: (0, i, 0)),
            pl.BlockSpec((_BLK, H), lambda i: (i, 0)),
            pl.BlockSpec((1, 1, _BLK), lambda i: (i, 0, 0)),
        ],
        out_specs=pl.BlockSpec((G, H), lambda i: (0, 0)),
        out_shape=jax.ShapeDtypeStruct((G, H), jnp.float32),
        scratch_shapes=[
            pltpu.VMEM((G, H), jnp.float32),
            pltpu.VMEM((G, H), jnp.float32),
        ],
    )(p, hroot2, batch3)


def kernel(x, edge_index, batch, W_rel1, b_rel1, W_root1,
           W_rel2, b_rel2, W_root2):
    src_flat = edge_index[0]
    dst = edge_index[1]
    batch3 = batch.reshape(N // _BLK, 1, _BLK)

    xroot1 = _tc_root(x, W_root1, b_rel1.reshape(1, H))
    p1 = _sc_aggregate(x, src_flat, dst)
    hmid, hrel2 = _tc_mid(p1, xroot1, W_rel1, W_rel2)
    hroot2 = _tc_root(hmid, W_root2, b_rel2.reshape(1, H))
    p2 = _sc_aggregate(hrel2, src_flat, dst)
    return _tc_pool(p2, hroot2, batch3)
